# baseline (speedup 1.0000x reference)
import functools

import jax
import jax.numpy as jnp
from jax import lax
from jax.experimental import pallas as pl
from jax.experimental.pallas import tpu as pltpu
from jax.experimental.pallas import tpu_sc as plsc

_CB = 32768


def _tc_body(tt_ref, w_ref, b_ref, out0_ref, out1_ref):
    t = jnp.tanh(tt_ref[...])
    acc = lax.dot_general(w_ref[...], t, (((1,), (0,)), ((), ())),
                          preferred_element_type=jnp.float32)
    out0_ref[...] = acc[0, :] + b_ref[0]
    out1_ref[...] = acc[1, :] + b_ref[1]


def _tc_half(table_t, lin_w, lin_b, blk0, nblk, width):
    D = table_t.shape[0]
    return pl.pallas_call(
        _tc_body,
        grid=(nblk,),
        in_specs=[
            pl.BlockSpec((D, _CB), lambda i: (0, i + blk0)),
            pl.BlockSpec((2, D), lambda i: (0, 0)),
            pl.BlockSpec(memory_space=pltpu.SMEM),
        ],
        out_specs=[
            pl.BlockSpec((_CB,), lambda i: (i,)),
            pl.BlockSpec((_CB,), lambda i: (i,)),
        ],
        out_shape=[
            jax.ShapeDtypeStruct((width,), jnp.float32),
            jax.ShapeDtypeStruct((width,), jnp.float32),
        ],
    )(table_t, lin_w, lin_b)


def _make_sc_gather(W, B):
    info = plsc.get_sparse_core_info()
    NC, NS, L = info.num_cores, info.num_subcores, info.num_lanes
    NW = NC * NS
    assert B % (8 * NW) == 0
    b_per_w = B // NW
    mesh = plsc.VectorSubcoreMesh(core_axis_name="c", subcore_axis_name="s")

    @functools.partial(
        pl.kernel,
        mesh=mesh,
        compiler_params=pltpu.CompilerParams(use_tc_tiling_on_sc=False),
        out_type=jax.ShapeDtypeStruct((2, B), jnp.float32),
        scratch_types=[
            pltpu.VMEM((b_per_w,), jnp.int32),
            pltpu.VMEM((2, b_per_w), jnp.float32),
            pltpu.SemaphoreType.DMA,
        ],
    )
    def sc_k(idx_hbm, l0_hbm, l1_hbm, out_hbm, idx_v, g_v, sem):
        wid = lax.axis_index("s") * NC + lax.axis_index("c")
        base = wid * b_per_w
        pltpu.sync_copy(idx_hbm.at[pl.ds(base, b_per_w)], idx_v)

        def clamp(g, c):
            iv = idx_v[pl.ds(g * L, L)]
            iv = jnp.minimum(jnp.maximum(iv, 0), W - 1)
            idx_v[pl.ds(g * L, L)] = iv
            return c

        lax.fori_loop(0, b_per_w // L, clamp, 0)
        pltpu.async_copy(l0_hbm.at[idx_v], g_v.at[0], sem).wait()
        pltpu.async_copy(l1_hbm.at[idx_v], g_v.at[1], sem).wait()
        pltpu.sync_copy(g_v, out_hbm.at[:, pl.ds(base, b_per_w)])

    return sc_k


def kernel(state_index, emb_table, lin_w, lin_b):
    V, D = emb_table.shape
    B = state_index.shape[0]
    idx = state_index.astype(jnp.int32)
    table_t = emb_table.T

    nblk_a = 16
    split = nblk_a * _CB
    nblk_b = pl.cdiv(V - split, _CB)
    width_b = V - split

    l0a, l1a = _tc_half(table_t, lin_w, lin_b, 0, nblk_a, split)
    l0b, l1b = _tc_half(table_t, lin_w, lin_b, nblk_a, nblk_b, width_b)

    ga = _make_sc_gather(split, B)(idx, l0a, l1a)
    gb = _make_sc_gather(width_b, B)(idx - split, l0b, l1b)

    sel = (idx < split)[:, None]
    return jnp.where(sel, ga.T, gb.T)


# speedup vs baseline: 2.4927x; 2.4927x over previous
import functools

import jax
import jax.numpy as jnp
from jax import lax
from jax.experimental import pallas as pl
from jax.experimental.pallas import tpu as pltpu
from jax.experimental.pallas import tpu_sc as plsc


def _tc_body(tt_ref, w_ref, b_ref, out0_ref, out1_ref):
    t = jnp.tanh(tt_ref[...])
    acc = lax.dot_general(w_ref[...], t, (((1,), (0,)), ((), ())),
                          preferred_element_type=jnp.float32)
    out0_ref[...] = acc[0, :] + b_ref[0]
    out1_ref[...] = acc[1, :] + b_ref[1]


def _make_sc_gather(V, B):
    info = plsc.get_sparse_core_info()
    NC, NS = info.num_cores, info.num_subcores
    NW = NC * NS
    assert B % (8 * NW) == 0
    b_per_w = B // NW
    mesh = plsc.VectorSubcoreMesh(core_axis_name="c", subcore_axis_name="s")

    @functools.partial(
        pl.kernel,
        mesh=mesh,
        compiler_params=pltpu.CompilerParams(use_tc_tiling_on_sc=False),
        out_type=jax.ShapeDtypeStruct((2, B), jnp.float32),
        scratch_types=[
            pltpu.VMEM((b_per_w,), jnp.int32),
            pltpu.VMEM((2, b_per_w), jnp.float32),
            pltpu.SemaphoreType.DMA,
        ],
    )
    def sc_k(idx_hbm, l0_hbm, l1_hbm, out_hbm, idx_v, g_v, sem):
        wid = lax.axis_index("s") * NC + lax.axis_index("c")
        base = wid * b_per_w
        pltpu.sync_copy(idx_hbm.at[pl.ds(base, b_per_w)], idx_v)
        pltpu.async_copy(l0_hbm.at[idx_v], g_v.at[0], sem).wait()
        pltpu.async_copy(l1_hbm.at[idx_v], g_v.at[1], sem).wait()
        pltpu.sync_copy(g_v, out_hbm.at[:, pl.ds(base, b_per_w)])

    return sc_k


def kernel(state_index, emb_table, lin_w, lin_b):
    V, D = emb_table.shape
    B = state_index.shape[0]
    idx = state_index.astype(jnp.int32)
    table_t = emb_table.T

    CB = 32768
    grid = pl.cdiv(V, CB)
    l0, l1 = pl.pallas_call(
        _tc_body,
        grid=(grid,),
        in_specs=[
            pl.BlockSpec((D, CB), lambda i: (0, i)),
            pl.BlockSpec((2, D), lambda i: (0, 0)),
            pl.BlockSpec(memory_space=pltpu.SMEM),
        ],
        out_specs=[
            pl.BlockSpec((CB,), lambda i: (i,)),
            pl.BlockSpec((CB,), lambda i: (i,)),
        ],
        out_shape=[
            jax.ShapeDtypeStruct((V,), jnp.float32),
            jax.ShapeDtypeStruct((V,), jnp.float32),
        ],
    )(table_t, lin_w, lin_b)

    out = _make_sc_gather(V, B)(idx, l0, l1)
    return out.T
